# Initial kernel scaffold; baseline (speedup 1.0000x reference)
import jax
import jax.numpy as jnp
from jax.experimental import pallas as pl


def kernel(x, edge_index, W1, as1, ad1, b1, W2, as2, ad2, b2, W3, as3, ad3, b3, W4, as4, ad4, b4, fcW, fcb):
    raise NotImplementedError("write your pallas kernel here")


# speedup vs baseline: 1.0145x; 1.0145x over previous
import functools
import jax
import jax.numpy as jnp
import numpy as np
from jax.experimental import pallas as pl
from jax.experimental.pallas import tpu as pltpu

N_NODES = 10000
HEADS = 8
CH = 32
FEAT = HEADS * CH
ROW_BLK = 1000


def _proj_body(apply_act, x_ref, w_ref, amat_ref, bprev_ref, h_ref, al_ref):
    x = x_ref[...]
    if apply_act:
        x = x + bprev_ref[...]
        x = jnp.where(x > 0, x, 0.2 * x)
    h = jnp.dot(x, w_ref[...], preferred_element_type=jnp.float32)
    h_ref[...] = h
    al_ref[...] = jnp.dot(h, amat_ref[...], preferred_element_type=jnp.float32)


def _project(x, W, amat, bprev, apply_act):
    grid = (N_NODES // ROW_BLK,)
    h, al = pl.pallas_call(
        functools.partial(_proj_body, apply_act),
        grid=grid,
        in_specs=[
            pl.BlockSpec((ROW_BLK, FEAT), lambda i: (i, 0)),
            pl.BlockSpec((FEAT, FEAT), lambda i: (0, 0)),
            pl.BlockSpec((FEAT, 2 * HEADS), lambda i: (0, 0)),
            pl.BlockSpec((1, FEAT), lambda i: (0, 0)),
        ],
        out_specs=[
            pl.BlockSpec((ROW_BLK, FEAT), lambda i: (i, 0)),
            pl.BlockSpec((ROW_BLK, 2 * HEADS), lambda i: (i, 0)),
        ],
        out_shape=[
            jax.ShapeDtypeStruct((N_NODES, FEAT), jnp.float32),
            jax.ShapeDtypeStruct((N_NODES, 2 * HEADS), jnp.float32),
        ],
    )(x, W, amat, bprev)
    return h, al[:, :HEADS], al[:, HEADS:]


def _fc_body(v_ref, b4_ref, w_ref, fcb_ref, o_ref):
    v = v_ref[...] + b4_ref[...]
    v = jnp.where(v > 0, v, 0.2 * v)
    acc = fcb_ref[...]
    for i in range(5):
        acc = acc + jnp.dot(v[i:i + 1, :], w_ref[i], preferred_element_type=jnp.float32)
    o_ref[...] = acc


def _edge_phase(h, asrc, adst, src, dst):
    e = asrc[src] + adst[dst]
    e = jnp.where(e > 0, e, 0.2 * e)
    emax = jax.ops.segment_max(e, dst, num_segments=N_NODES)
    emax = jnp.where(jnp.isfinite(emax), emax, 0.0)
    ex = jnp.exp(e - emax[dst])
    denom = jax.ops.segment_sum(ex, dst, num_segments=N_NODES)
    alpha = ex / (denom[dst] + 1e-16)
    out = jax.ops.segment_sum(
        h[src].reshape(-1, HEADS, CH) * alpha[:, :, None], dst, num_segments=N_NODES
    )
    return out.reshape(N_NODES, FEAT)


def kernel(x, edge_index, W1, as1, ad1, b1, W2, as2, ad2, b2, W3, as3, ad3, b3,
           W4, as4, ad4, b4, fcW, fcb):
    src = edge_index[0]
    dst = edge_index[1]

    def pack_a(a_s, a_d):
        m = jnp.zeros((FEAT, 2 * HEADS), jnp.float32)
        hh = jnp.arange(FEAT) // CH
        m = m.at[jnp.arange(FEAT), hh].set(a_s.reshape(FEAT))
        m = m.at[jnp.arange(FEAT), HEADS + hh].set(a_d.reshape(FEAT))
        return m

    zeros_b = jnp.zeros((1, FEAT), jnp.float32)
    layers = [
        (W1, pack_a(as1, ad1), zeros_b, False),
        (W2, pack_a(as2, ad2), b1.reshape(1, FEAT), True),
        (W3, pack_a(as3, ad3), b2.reshape(1, FEAT), True),
        (W4, pack_a(as4, ad4), b3.reshape(1, FEAT), True),
    ]

    cur = x
    for W, amat, bprev, act in layers:
        h, a_s, a_d = _project(cur, W, amat, bprev, act)
        cur = _edge_phase(h, a_s, a_d, src, dst)

    v5 = cur[:5]
    out = pl.pallas_call(
        _fc_body,
        out_shape=jax.ShapeDtypeStruct((1, 128), jnp.float32),
    )(v5, jnp.broadcast_to(b4.reshape(1, FEAT), (5, FEAT)),
      fcW.reshape(5, FEAT, 128), fcb.reshape(1, 128))
    return out.reshape(128)
